# baseline (speedup 1.0000x reference)
import functools

import jax
import jax.numpy as jnp
from jax import lax
from jax.experimental import pallas as pl
from jax.experimental.pallas import tpu as pltpu
from jax.experimental.pallas import tpu_sc as plsc

_B, _S, _D = 2, 2048, 1024
_C = 64
_K = 8
_LB = 128
_NEG = -1e9
_BT = 256
_NB = _S // _BT
_NT = 2 * _B * _NB
_SCALE = 1.0 / 32.0
_HI = jax.lax.Precision.HIGHEST


def _bdot(a_bf, b_bf):
    return jax.lax.dot_general(a_bf, b_bf, (((1,), (1,)), ((), ())),
                               preferred_element_type=jnp.float32)


def _front_body(h_ref, w_ref, b_ref, ms_ref, sp_ref, v_ref, pv_ref,
                kc, vc, pkc, pvc):
    i = pl.program_id(1)
    t0 = i * _BT

    @pl.when(i == 0)
    def _reset():
        kc[...] = jnp.zeros_like(kc)
        vc[...] = jnp.zeros_like(vc)
        pkc[...] = jnp.zeros_like(pkc)
        pvc[...] = jnp.zeros_like(pvc)

    x = h_ref[...]
    xb = x.astype(jnp.bfloat16)
    q = _bdot(xb, w_ref[0]) + b_ref[0]
    k = _bdot(xb, w_ref[1]) + b_ref[1]
    v = _bdot(xb, w_ref[2]) + b_ref[2]

    rr = jax.lax.broadcasted_iota(jnp.int32, (_BT, _BT), 0)
    cc = jax.lax.broadcasted_iota(jnp.int32, (_BT, _BT), 1)
    tri = ((rr // _C) == (cc // _C)) & (cc <= rr)
    tmat = jnp.where(tri, 1.0 / (1.0 + (rr % _C).astype(jnp.float32)), 0.0)
    ps = jax.lax.dot_general(tmat, x, (((1,), (0,)), ((), ())),
                             preferred_element_type=jnp.float32, precision=_HI)
    psb = ps.astype(jnp.bfloat16)
    pk = _bdot(psb, w_ref[3]) + b_ref[3]
    pv = _bdot(psb, w_ref[4]) + b_ref[4]

    kw = jnp.concatenate([kc[...], k], axis=0)
    pkw = jnp.concatenate([pkc[...], pk], axis=0)
    qb = q.astype(jnp.bfloat16)
    kwb = kw.astype(jnp.bfloat16)

    v_ref[0] = v.astype(jnp.bfloat16)
    pv_ref[0] = pv.astype(jnp.bfloat16)
    for half in range(2):
        r0 = half * _LB
        scores = jax.lax.dot_general(
            qb[r0:r0 + _LB], kwb[r0:r0 + 2 * _LB], (((1,), (1,)), ((), ())),
            preferred_element_type=jnp.float32) * _SCALE
        ii = jax.lax.broadcasted_iota(jnp.int32, (_LB, 2 * _LB), 0)
        jj = jax.lax.broadcasted_iota(jnp.int32, (_LB, 2 * _LB), 1)
        valid = (jj >= ii) & (jj < ii + _LB) & (jj + r0 + t0 >= _LB)
        ms_ref[half, :, :] = jnp.where(valid, scores, _NEG)

        kprev = kw[r0 + _LB - 1:r0 + 2 * _LB - 1]
        pkprev = pkw[r0 + _LB - 1:r0 + 2 * _LB - 1]
        seq = jnp.sum(q[r0:r0 + _LB] * kprev, axis=1) * _SCALE
        ph = jnp.sum(q[r0:r0 + _LB] * pkprev, axis=1) * _SCALE
        sp_ref[0, half, 0, :] = seq
        sp_ref[1, half, 0, :] = ph

    kc[...] = k[_BT - _LB:]
    vc[...] = v[_BT - _LB:]
    pkc[...] = pk[_BT - _LB:]
    pvc[...] = pv[_BT - _LB:]


def _sc_topk(ms_hbm, st_hbm, tile_v, st_v):
    wid = lax.axis_index("s") * 2 + lax.axis_index("c")
    pltpu.sync_copy(ms_hbm.at[wid], tile_v)
    lanes = lax.iota(jnp.int32, 16)
    neg = jnp.full((16,), _NEG, jnp.float32)

    def _sort_desc(x):
        return plsc.sort_key_val(x, x, descending=True)[0]

    def tok(r, carry):
        ph = r % 16
        base = r * 257 - ph
        phv = jnp.full((16,), ph, jnp.int32)
        vs = [tile_v[pl.ds(base + 16 * g, 16)] for g in range(9)]
        vs[0] = jnp.where(lanes >= phv, vs[0], neg)
        vs[8] = jnp.where(lanes < phv, vs[8], neg)
        ss = [_sort_desc(x) for x in vs]
        while len(ss) > 1:
            nxt = []
            for a, b in zip(ss[::2], ss[1::2]):
                nxt.append(_sort_desc(jnp.maximum(lax.rev(a, (0,)), b)))
            if len(ss) % 2:
                nxt.append(ss[-1])
            ss = nxt
        st_v[r, :] = ss[0]
        return carry

    lax.fori_loop(0, _LB, tok, 0)
    pltpu.sync_copy(st_v, st_hbm.at[wid])


def _back_body(ms_ref, sp_ref, st_ref,
               vp_ref, v0_ref, v1_ref, pvp_ref, pv0_ref, pv1_ref, o_ref):
    vw = jnp.concatenate([vp_ref[0, 0], v0_ref[0, 0], v1_ref[0, 0]], axis=0)
    pvw = jnp.concatenate([pvp_ref[0, 0], pv0_ref[0, 0], pv1_ref[0, 0]], axis=0)
    for half in range(2):
        r0 = half * _LB
        masked = ms_ref[half]
        seq = sp_ref[0, half, 0, :]
        ph = sp_ref[1, half, 0, :]
        thr = st_ref[half, :, 7]
        m1 = st_ref[half, :, 0]
        keep = masked >= thr[:, None]
        m10 = jnp.maximum(jnp.maximum(seq, ph), m1)
        eb = jnp.where(keep, jnp.exp(masked - m10[:, None]), 0.0)
        es = jnp.exp(seq - m10)
        ep = jnp.exp(ph - m10)
        z = es + ep + jnp.sum(eb, axis=1)

        vw_h = vw[r0:r0 + 2 * _LB]
        pvw_h = pvw[r0:r0 + 2 * _LB]
        acc = jax.lax.dot_general(eb.astype(jnp.bfloat16), vw_h,
                                  (((1,), (0,)), ((), ())),
                                  preferred_element_type=jnp.float32)
        acc = acc + es[:, None] * vw_h[_LB - 1:2 * _LB - 1].astype(jnp.float32)
        acc = acc + ep[:, None] * pvw_h[_LB - 1:2 * _LB - 1].astype(jnp.float32)
        out = acc / z[:, None]

        i = pl.program_id(1)
        tvec = i * _BT + r0 + jax.lax.broadcasted_iota(jnp.int32, (_LB, 1), 0)
        o_ref[0, r0:r0 + _LB] = jnp.where(tvec > 0, out, 0.0)


def kernel(h, Wq, bq, Wk, bk, Wv, bv, Wpk, bpk, Wpv, bpv):
    hf = h.reshape(_B * _S, _D)
    wall = jnp.stack([Wq, Wk, Wv, Wpk, Wpv]).astype(jnp.bfloat16)
    ball = jnp.stack([bq, bk, bv, bpk, bpv]).reshape(5, 1, _D)

    masked, sp, vfull, pvfull = pl.pallas_call(
        _front_body,
        grid=(_B, _NB),
        in_specs=[
            pl.BlockSpec((_BT, _D), lambda b, i: (b * _NB + i, 0)),
            pl.BlockSpec((5, _D, _D), lambda b, i: (0, 0, 0)),
            pl.BlockSpec((5, 1, _D), lambda b, i: (0, 0, 0)),
        ],
        out_specs=[
            pl.BlockSpec((2, _LB, 2 * _LB), lambda b, i: (b * _NB + i, 0, 0)),
            pl.BlockSpec((2, 2, 1, _LB), lambda b, i: (0, b * _NB + i, 0, 0)),
            pl.BlockSpec((1, _BT, _D), lambda b, i: (b * _NB + i, 0, 0)),
            pl.BlockSpec((1, _BT, _D), lambda b, i: (b * _NB + i, 0, 0)),
        ],
        out_shape=[
            jax.ShapeDtypeStruct((_NT, _LB, 2 * _LB), jnp.float32),
            jax.ShapeDtypeStruct((2, _NT, 1, _LB), jnp.float32),
            jax.ShapeDtypeStruct((_B * _NB, _BT, _D), jnp.bfloat16),
            jax.ShapeDtypeStruct((_B * _NB, _BT, _D), jnp.bfloat16),
        ],
        scratch_shapes=[pltpu.VMEM((_LB, _D), jnp.float32)] * 4,
    )(hf, wall, ball)

    sc_call = functools.partial(
        pl.kernel,
        out_type=jax.ShapeDtypeStruct((_NT, _LB, 16), jnp.float32),
        mesh=plsc.VectorSubcoreMesh(core_axis_name="c", subcore_axis_name="s"),
        scratch_types=[pltpu.VMEM((_LB * 2 * _LB,), jnp.float32),
                       pltpu.VMEM((_LB, 16), jnp.float32)],
        compiler_params=pltpu.CompilerParams(needs_layout_passes=False),
    )(_sc_topk)
    st = sc_call(masked.reshape(_NT, _LB * 2 * _LB))

    v3 = vfull.reshape(_B, _S // _LB, _LB, _D)
    pv3 = pvfull.reshape(_B, _S // _LB, _LB, _D)

    def vspec(off):
        if off < 0:
            return pl.BlockSpec(
                (1, 1, _LB, _D),
                lambda b, i: (b, jnp.maximum(2 * i - 1, 0), 0, 0))
        return pl.BlockSpec(
            (1, 1, _LB, _D), lambda b, i, off=off: (b, 2 * i + off, 0, 0))

    out = pl.pallas_call(
        _back_body,
        grid=(_B, _NB),
        in_specs=[
            pl.BlockSpec((2, _LB, 2 * _LB), lambda b, i: (b * _NB + i, 0, 0)),
            pl.BlockSpec((2, 2, 1, _LB), lambda b, i: (0, b * _NB + i, 0, 0)),
            pl.BlockSpec((2, _LB, 16), lambda b, i: (b * _NB + i, 0, 0)),
            vspec(-1), vspec(0), vspec(1),
            vspec(-1), vspec(0), vspec(1),
        ],
        out_specs=pl.BlockSpec((1, _BT, _D), lambda b, i: (b, i, 0)),
        out_shape=jax.ShapeDtypeStruct((_B, _S, _D), jnp.float32),
    )(masked, sp, st, v3, v3, v3, pv3, pv3, pv3)
    return out


# speedup vs baseline: 1.0512x; 1.0512x over previous
import functools

import jax
import jax.numpy as jnp
from jax import lax
from jax.experimental import pallas as pl
from jax.experimental.pallas import tpu as pltpu
from jax.experimental.pallas import tpu_sc as plsc

_B, _S, _D = 2, 2048, 1024
_C = 64
_K = 8
_LB = 128
_NEG = -1e9
_BT = 256
_NB = _S // _BT
_NT = 2 * _B * _NB
_SCALE = 1.0 / 32.0
_HI = jax.lax.Precision.HIGHEST


def _bdot(a_bf, b_bf):
    return jax.lax.dot_general(a_bf, b_bf, (((1,), (1,)), ((), ())),
                               preferred_element_type=jnp.float32)


def _front_body(h_ref, w_ref, b_ref, t_ref, ms_ref, sp_ref, v_ref, pv_ref,
                kc, pkc):
    i = pl.program_id(1)
    t0 = i * _BT

    x = h_ref[...]
    xb = x.astype(jnp.bfloat16)
    q = _bdot(xb, w_ref[0]) + b_ref[0]
    k = _bdot(xb, w_ref[1]) + b_ref[1]
    v = _bdot(xb, w_ref[2]) + b_ref[2]

    xlo = (x - xb.astype(jnp.float32)).astype(jnp.bfloat16)

    def _tdot(t_bf, y_bf):
        return jax.lax.dot_general(t_bf, y_bf, (((1,), (0,)), ((), ())),
                                   preferred_element_type=jnp.float32)

    ps = (_tdot(t_ref[0], xb) + _tdot(t_ref[0], xlo) + _tdot(t_ref[1], xb))
    psb = ps.astype(jnp.bfloat16)
    pk = _bdot(psb, w_ref[3]) + b_ref[3]
    pv = _bdot(psb, w_ref[4]) + b_ref[4]

    kw = jnp.concatenate([kc[...], k], axis=0)
    pkw = jnp.concatenate([pkc[...], pk], axis=0)
    qb = q.astype(jnp.bfloat16)
    kwb = kw.astype(jnp.bfloat16)

    v_ref[0] = v.astype(jnp.bfloat16)
    pv_ref[0] = pv.astype(jnp.bfloat16)
    for half in range(2):
        r0 = half * _LB
        scores = jax.lax.dot_general(
            qb[r0:r0 + _LB], kwb[r0:r0 + 2 * _LB], (((1,), (1,)), ((), ())),
            preferred_element_type=jnp.float32) * _SCALE
        ii = jax.lax.broadcasted_iota(jnp.int32, (_LB, 2 * _LB), 0)
        jj = jax.lax.broadcasted_iota(jnp.int32, (_LB, 2 * _LB), 1)
        valid = (jj >= ii) & (jj < ii + _LB) & (jj + r0 + t0 >= _LB)
        ms_ref[half, :, :] = jnp.where(valid, scores, _NEG)

        kprev = kw[r0 + _LB - 1:r0 + 2 * _LB - 1]
        pkprev = pkw[r0 + _LB - 1:r0 + 2 * _LB - 1]
        seq = jnp.sum(q[r0:r0 + _LB] * kprev, axis=1) * _SCALE
        ph = jnp.sum(q[r0:r0 + _LB] * pkprev, axis=1) * _SCALE
        sp_ref[0, half, 0, :] = seq
        sp_ref[1, half, 0, :] = ph

    kc[...] = k[_BT - _LB:]
    pkc[...] = pk[_BT - _LB:]


def _sc_topk(ms_hbm, st_hbm, tile_v, st_v):
    wid = lax.axis_index("s") * 2 + lax.axis_index("c")
    pltpu.sync_copy(ms_hbm.at[wid], tile_v)
    lanes = lax.iota(jnp.int32, 16)
    neg = jnp.full((16,), _NEG, jnp.float32)

    def _sort_desc(x):
        return plsc.sort_key_val(x, x, descending=True)[0]

    def tok(r, carry):
        ph = r % 16
        base = r * 257 - ph
        phv = jnp.full((16,), ph, jnp.int32)
        vs = [tile_v[pl.ds(base + 16 * g, 16)] for g in range(9)]
        vs[0] = jnp.where(lanes >= phv, vs[0], neg)
        vs[8] = jnp.where(lanes < phv, vs[8], neg)
        ss = [_sort_desc(x) for x in vs]
        while len(ss) > 1:
            nxt = []
            for a, b in zip(ss[::2], ss[1::2]):
                nxt.append(_sort_desc(jnp.maximum(lax.rev(a, (0,)), b)))
            if len(ss) % 2:
                nxt.append(ss[-1])
            ss = nxt
        st_v[r, :] = ss[0]
        return carry

    lax.fori_loop(0, _LB, tok, 0)
    pltpu.sync_copy(st_v, st_hbm.at[wid])


def _back_body(ms_ref, sp_ref, st_ref,
               vp_ref, v0_ref, v1_ref, pvp_ref, pv0_ref, pv1_ref, o_ref):
    vw = jnp.concatenate([vp_ref[0, 0], v0_ref[0, 0], v1_ref[0, 0]], axis=0)
    pvw = jnp.concatenate([pvp_ref[0, 0], pv0_ref[0, 0], pv1_ref[0, 0]], axis=0)
    for half in range(2):
        r0 = half * _LB
        masked = ms_ref[half]
        seq = sp_ref[0, half, 0, :]
        ph = sp_ref[1, half, 0, :]
        thr = st_ref[half, :, 7]
        m1 = st_ref[half, :, 0]
        keep = masked >= thr[:, None]
        m10 = jnp.maximum(jnp.maximum(seq, ph), m1)
        eb = jnp.where(keep, jnp.exp(masked - m10[:, None]), 0.0)
        es = jnp.exp(seq - m10)
        ep = jnp.exp(ph - m10)
        z = es + ep + jnp.sum(eb, axis=1)

        vw_h = vw[r0:r0 + 2 * _LB]
        pvw_h = pvw[r0:r0 + 2 * _LB]
        acc = jax.lax.dot_general(eb.astype(jnp.bfloat16), vw_h,
                                  (((1,), (0,)), ((), ())),
                                  preferred_element_type=jnp.float32)
        acc = acc + es[:, None] * vw_h[_LB - 1:2 * _LB - 1].astype(jnp.float32)
        acc = acc + ep[:, None] * pvw_h[_LB - 1:2 * _LB - 1].astype(jnp.float32)
        out = acc / z[:, None]

        i = pl.program_id(1)
        tvec = i * _BT + r0 + jax.lax.broadcasted_iota(jnp.int32, (_LB, 1), 0)
        o_ref[0, r0:r0 + _LB] = jnp.where(tvec > 0, out, 0.0)


def kernel(h, Wq, bq, Wk, bk, Wv, bv, Wpk, bpk, Wpv, bpv):
    hf = h.reshape(_B * _S, _D)
    wall = jnp.stack([Wq, Wk, Wv, Wpk, Wpv]).astype(jnp.bfloat16)
    ball = jnp.stack([bq, bk, bv, bpk, bpv]).reshape(5, 1, _D)
    rr = jnp.arange(_BT)[:, None]
    cc = jnp.arange(_BT)[None, :]
    tri = ((rr // _C) == (cc // _C)) & (cc <= rr)
    tmat = jnp.where(tri, 1.0 / (1.0 + (rr % _C)), 0.0).astype(jnp.float32)
    tmat_hi = tmat.astype(jnp.bfloat16)
    tmat_lo = (tmat - tmat_hi.astype(jnp.float32)).astype(jnp.bfloat16)
    tpair = jnp.stack([tmat_hi, tmat_lo])

    masked, sp, vfull, pvfull = pl.pallas_call(
        _front_body,
        grid=(_B, _NB),
        in_specs=[
            pl.BlockSpec((_BT, _D), lambda b, i: (b * _NB + i, 0)),
            pl.BlockSpec((5, _D, _D), lambda b, i: (0, 0, 0)),
            pl.BlockSpec((5, 1, _D), lambda b, i: (0, 0, 0)),
            pl.BlockSpec((2, _BT, _BT), lambda b, i: (0, 0, 0)),
        ],
        out_specs=[
            pl.BlockSpec((2, _LB, 2 * _LB), lambda b, i: (b * _NB + i, 0, 0)),
            pl.BlockSpec((2, 2, 1, _LB), lambda b, i: (0, b * _NB + i, 0, 0)),
            pl.BlockSpec((1, _BT, _D), lambda b, i: (b * _NB + i, 0, 0)),
            pl.BlockSpec((1, _BT, _D), lambda b, i: (b * _NB + i, 0, 0)),
        ],
        out_shape=[
            jax.ShapeDtypeStruct((_NT, _LB, 2 * _LB), jnp.float32),
            jax.ShapeDtypeStruct((2, _NT, 1, _LB), jnp.float32),
            jax.ShapeDtypeStruct((_B * _NB, _BT, _D), jnp.bfloat16),
            jax.ShapeDtypeStruct((_B * _NB, _BT, _D), jnp.bfloat16),
        ],
        scratch_shapes=[pltpu.VMEM((_LB, _D), jnp.float32)] * 2,
    )(hf, wall, ball, tpair)

    sc_call = functools.partial(
        pl.kernel,
        out_type=jax.ShapeDtypeStruct((_NT, _LB, 16), jnp.float32),
        mesh=plsc.VectorSubcoreMesh(core_axis_name="c", subcore_axis_name="s"),
        scratch_types=[pltpu.VMEM((_LB * 2 * _LB,), jnp.float32),
                       pltpu.VMEM((_LB, 16), jnp.float32)],
        compiler_params=pltpu.CompilerParams(needs_layout_passes=False),
    )(_sc_topk)
    st = sc_call(masked.reshape(_NT, _LB * 2 * _LB))

    v3 = vfull.reshape(_B, _S // _LB, _LB, _D)
    pv3 = pvfull.reshape(_B, _S // _LB, _LB, _D)

    def vspec(off):
        if off < 0:
            return pl.BlockSpec(
                (1, 1, _LB, _D),
                lambda b, i: (b, jnp.maximum(2 * i - 1, 0), 0, 0))
        return pl.BlockSpec(
            (1, 1, _LB, _D), lambda b, i, off=off: (b, 2 * i + off, 0, 0))

    out = pl.pallas_call(
        _back_body,
        grid=(_B, _NB),
        in_specs=[
            pl.BlockSpec((2, _LB, 2 * _LB), lambda b, i: (b * _NB + i, 0, 0)),
            pl.BlockSpec((2, 2, 1, _LB), lambda b, i: (0, b * _NB + i, 0, 0)),
            pl.BlockSpec((2, _LB, 16), lambda b, i: (b * _NB + i, 0, 0)),
            vspec(-1), vspec(0), vspec(1),
            vspec(-1), vspec(0), vspec(1),
        ],
        out_specs=pl.BlockSpec((1, _BT, _D), lambda b, i: (b, i, 0)),
        out_shape=jax.ShapeDtypeStruct((_B, _S, _D), jnp.float32),
    )(masked, sp, st, v3, v3, v3, pv3, pv3, pv3)
    return out
